# baseline (speedup 1.0000x reference)
import dataclasses
import functools

import jax
import jax.numpy as jnp
from jax import lax
from jax.experimental import pallas as pl
from jax.experimental.pallas import tpu as pltpu
from jax.experimental.pallas import tpu_sc as plsc

N = 10000
D = 128
E = 320000
NCORE = 2
NSUB = 16
NW = NCORE * NSUB
EPW = E // NW
C = 80
NCHUNK = EPW // C
NP = 10240
RPT = NP // NSUB

CA = 128
EP = 327680
EPWA = EP // NW
NCA = EPWA // CA

_mesh = plsc.VectorSubcoreMesh(core_axis_name="c", subcore_axis_name="s")

_sc_params = pltpu.CompilerParams()
if "needs_layout_passes" in pltpu.CompilerParams.__dataclass_fields__:
    _sc_params = dataclasses.replace(_sc_params, needs_layout_passes=False)


def _zero_fill(ref, rows, cols):
    z = jnp.zeros((16,), jnp.float32)

    @pl.loop(0, rows)
    def _(r):
        @pl.loop(0, cols, step=16)
        def _(j):
            ref[r, pl.ds(j, 16)] = z


@functools.partial(
    pl.kernel,
    mesh=_mesh,
    out_type=jax.ShapeDtypeStruct((NW * NP,), jnp.float32),
    compiler_params=_sc_params,
    scratch_types=[
        pltpu.VMEM((EPW,), jnp.int32),
        pltpu.VMEM((NP,), jnp.float32),
    ],
)
def _sc_degree(dst_hbm, out_hbm, dst_v, hist):
    cid = lax.axis_index("c")
    sid = lax.axis_index("s")
    wid = cid * NSUB + sid

    z = jnp.zeros((16,), jnp.float32)

    @pl.loop(0, NP, step=16)
    def _(j):
        hist[pl.ds(j, 16)] = z

    pltpu.sync_copy(dst_hbm.at[pl.ds(wid * EPW, EPW)], dst_v)

    one = jnp.ones((16,), jnp.float32)

    @pl.loop(0, EPW, step=16)
    def _(i):
        dv = dst_v[pl.ds(i, 16)]
        plsc.addupdate_scatter(hist, [dv], one)

    plsc.subcore_barrier()
    pltpu.sync_copy(hist, out_hbm.at[pl.ds(wid * NP, NP)])


@functools.partial(
    pl.kernel,
    mesh=_mesh,
    out_type=jax.ShapeDtypeStruct((NCORE, NP, D), jnp.float32),
    scratch_types=[
        pltpu.VMEM((CA,), jnp.int32),
        pltpu.VMEM((CA,), jnp.int32),
        pltpu.VMEM((CA, D), jnp.float32),
        pltpu.VMEM_SHARED((NP, D), jnp.float32),
        pltpu.SemaphoreType.DMA,
    ],
)
def _sc_aggregate(y_hbm, src_hbm, dst_hbm, out_hbm,
                  src_v, dst_v, rows_v, acc, sem):
    cid = lax.axis_index("c")
    sid = lax.axis_index("s")
    wid = cid * NSUB + sid

    _zero_fill(rows_v, 80, D)

    @pl.loop(0, RPT // 80)
    def _(k):
        pltpu.sync_copy(rows_v.at[pl.ds(0, 80)], acc.at[pl.ds(sid * RPT + k * 80, 80)])

    plsc.subcore_barrier()

    @pl.loop(0, NCA)
    def _(i):
        b = wid * EPWA + i * CA
        pltpu.sync_copy(src_hbm.at[pl.ds(b, CA)], src_v)
        pltpu.sync_copy(dst_hbm.at[pl.ds(b, CA)], dst_v)
        pltpu.async_copy(y_hbm.at[src_v], rows_v, sem).wait()
        pltpu.sync_copy(rows_v, acc.at[dst_v], add=True)

    plsc.subcore_barrier()
    pltpu.sync_copy(acc.at[pl.ds(sid * RPT, RPT)],
                    out_hbm.at[cid, pl.ds(sid * RPT, RPT)])


_BLK = 2000


def _tc_dis(degp):
    def body(d_ref, o_ref):
        deg = jnp.sum(d_ref[...], axis=0) + 1.0
        o_ref[...] = lax.rsqrt(deg)

    return pl.pallas_call(
        body,
        in_specs=[pl.BlockSpec((NW, NP // D, D), lambda: (0, 0, 0))],
        out_specs=pl.BlockSpec((NP // D, D), lambda: (0, 0)),
        out_shape=jax.ShapeDtypeStruct((NP // D, D), jnp.float32),
    )(degp)


def _tc_stage0(dis, x, W1):
    def body(dis_ref, x_ref, w_ref, o_ref):
        xw = jnp.dot(x_ref[...], w_ref[...], preferred_element_type=jnp.float32)
        o_ref[...] = xw * dis_ref[...]

    return pl.pallas_call(
        body,
        grid=(N // _BLK,),
        in_specs=[
            pl.BlockSpec((_BLK, 1), lambda i: (i, 0)),
            pl.BlockSpec((_BLK, D), lambda i: (i, 0)),
            pl.BlockSpec((D, D), lambda i: (0, 0)),
        ],
        out_specs=pl.BlockSpec((_BLK, D), lambda i: (i, 0)),
        out_shape=jax.ShapeDtypeStruct((N, D), jnp.float32),
    )(dis, x, W1)


def _tc_stage1(dis, agg, y1, b1r, W2):
    def body(dis_ref, a_ref, y_ref, b_ref, w_ref, o_ref):
        dis_b = dis_ref[...]
        h = dis_b * (a_ref[0] + a_ref[1] + y_ref[...]) + b_ref[...]
        h = jnp.maximum(h, 0.0)
        o_ref[...] = dis_b * jnp.dot(h, w_ref[...],
                                     preferred_element_type=jnp.float32)

    return pl.pallas_call(
        body,
        grid=(N // _BLK,),
        in_specs=[
            pl.BlockSpec((_BLK, 1), lambda i: (i, 0)),
            pl.BlockSpec((NCORE, _BLK, D), lambda i: (0, i, 0)),
            pl.BlockSpec((_BLK, D), lambda i: (i, 0)),
            pl.BlockSpec((1, D), lambda i: (0, 0)),
            pl.BlockSpec((D, D), lambda i: (0, 0)),
        ],
        out_specs=pl.BlockSpec((_BLK, D), lambda i: (i, 0)),
        out_shape=jax.ShapeDtypeStruct((N, D), jnp.float32),
    )(dis, agg, y1, b1r, W2)


def _tc_stage2(dis, agg, y2, b2r, fcWp, fcbr):
    def body(dis_ref, a_ref, y_ref, b_ref, w_ref, fb_ref, o_ref):
        h = dis_ref[...] * (a_ref[0] + a_ref[1] + y_ref[...]) + b_ref[...]
        o = jnp.dot(h, w_ref[...], preferred_element_type=jnp.float32)
        o = o + fb_ref[...]
        o_ref[...] = jnp.where(o >= 0.0, o, 0.01 * o)

    return pl.pallas_call(
        body,
        grid=(N // _BLK,),
        in_specs=[
            pl.BlockSpec((_BLK, 1), lambda i: (i, 0)),
            pl.BlockSpec((NCORE, _BLK, D), lambda i: (0, i, 0)),
            pl.BlockSpec((_BLK, D), lambda i: (i, 0)),
            pl.BlockSpec((1, D), lambda i: (0, 0)),
            pl.BlockSpec((D, D), lambda i: (0, 0)),
            pl.BlockSpec((1, D), lambda i: (0, 0)),
        ],
        out_specs=pl.BlockSpec((_BLK, D), lambda i: (i, 0)),
        out_shape=jax.ShapeDtypeStruct((N, D), jnp.float32),
    )(dis, agg, y2, b2r, fcWp, fcbr)


def kernel(x, edge_index, W1, b1, W2, b2, fc_W, fc_b):
    src = edge_index[0].astype(jnp.int32)
    dst = edge_index[1].astype(jnp.int32)
    b1r = b1.reshape(1, D)
    b2r = b2.reshape(1, D)
    fcWp = jnp.zeros((D, D), jnp.float32).at[:, :2].set(fc_W)
    fcbr = jnp.zeros((1, D), jnp.float32).at[0, :2].set(fc_b)

    pad = EP - E
    src1 = jnp.concatenate([src, jnp.zeros((pad,), jnp.int32)])
    dst1 = jnp.concatenate(
        [dst, N + (jnp.arange(pad, dtype=jnp.int32) % (NP - N))])

    degp = _sc_degree(dst).reshape(NW, NP // D, D)
    dis = _tc_dis(degp).reshape(NP, 1)[:N]

    y1 = _tc_stage0(dis, x, W1)
    agg1 = _sc_aggregate(y1, src1, dst1)[:, :N]
    y2 = _tc_stage1(dis, agg1, y1, b1r, W2)
    agg2 = _sc_aggregate(y2, src1, dst1)[:, :N]
    out = _tc_stage2(dis, agg2, y2, b2r, fcWp, fcbr)
    return out[:, :2]


# speedup vs baseline: 1.8164x; 1.8164x over previous
import dataclasses
import functools

import jax
import jax.numpy as jnp
from jax import lax
from jax.experimental import pallas as pl
from jax.experimental.pallas import tpu as pltpu
from jax.experimental.pallas import tpu_sc as plsc

N = 10000
D = 128
E = 320000
NCORE = 2
NSUB = 16
NW = NCORE * NSUB
EPW = E // NW
C = 80
NCHUNK = EPW // C
NP = 10240
RPT = NP // NSUB

CA = 80
EPWA = E // NW
NCA = EPWA // CA

_mesh = plsc.VectorSubcoreMesh(core_axis_name="c", subcore_axis_name="s")

_sc_params = pltpu.CompilerParams()
if "needs_layout_passes" in pltpu.CompilerParams.__dataclass_fields__:
    _sc_params = dataclasses.replace(_sc_params, needs_layout_passes=False)


def _zero_fill(ref, rows, cols):
    z = jnp.zeros((16,), jnp.float32)

    @pl.loop(0, rows)
    def _(r):
        @pl.loop(0, cols, step=16)
        def _(j):
            ref[r, pl.ds(j, 16)] = z


@functools.partial(
    pl.kernel,
    mesh=_mesh,
    out_type=jax.ShapeDtypeStruct((NW * NP,), jnp.float32),
    compiler_params=_sc_params,
    scratch_types=[
        pltpu.VMEM((EPW,), jnp.int32),
        pltpu.VMEM((NP,), jnp.float32),
    ],
)
def _sc_degree(dst_hbm, out_hbm, dst_v, hist):
    cid = lax.axis_index("c")
    sid = lax.axis_index("s")
    wid = cid * NSUB + sid

    z = jnp.zeros((16,), jnp.float32)

    @pl.loop(0, NP, step=16)
    def _(j):
        hist[pl.ds(j, 16)] = z

    pltpu.sync_copy(dst_hbm.at[pl.ds(wid * EPW, EPW)], dst_v)

    one = jnp.ones((16,), jnp.float32)

    @pl.loop(0, EPW, step=16)
    def _(i):
        dv = dst_v[pl.ds(i, 16)]
        plsc.addupdate_scatter(hist, [dv], one)

    plsc.subcore_barrier()
    pltpu.sync_copy(hist, out_hbm.at[pl.ds(wid * NP, NP)])


@functools.partial(
    pl.kernel,
    mesh=_mesh,
    out_type=jax.ShapeDtypeStruct((NCORE, NP, D), jnp.float32),
    scratch_types=[
        pltpu.VMEM((CA,), jnp.int32),
        pltpu.VMEM((CA,), jnp.int32),
        pltpu.VMEM((CA, D), jnp.float32),
        pltpu.VMEM_SHARED((NP, D), jnp.float32),
        pltpu.SemaphoreType.DMA,
    ],
)
def _sc_aggregate(y_hbm, src_hbm, dst_hbm, out_hbm,
                  src_v, dst_v, rows_v, acc, sem):
    cid = lax.axis_index("c")
    sid = lax.axis_index("s")
    wid = cid * NSUB + sid

    _zero_fill(rows_v, CA, D)

    @pl.loop(0, RPT // CA)
    def _(k):
        pltpu.sync_copy(rows_v, acc.at[pl.ds(sid * RPT + k * CA, CA)])

    plsc.subcore_barrier()

    @pl.loop(0, NCA)
    def _(i):
        b = wid * EPWA + i * CA
        pltpu.sync_copy(src_hbm.at[pl.ds(b, CA)], src_v)
        pltpu.sync_copy(dst_hbm.at[pl.ds(b, CA)], dst_v)
        pltpu.async_copy(y_hbm.at[src_v], rows_v, sem).wait()
        pltpu.sync_copy(rows_v, acc.at[dst_v], add=True)

    plsc.subcore_barrier()
    pltpu.sync_copy(acc.at[pl.ds(sid * RPT, RPT)],
                    out_hbm.at[cid, pl.ds(sid * RPT, RPT)])


_BLK = 2000


def _tc_dis(degp):
    def body(d_ref, o_ref):
        deg = jnp.sum(d_ref[...], axis=0) + 1.0
        o_ref[...] = lax.rsqrt(deg)

    return pl.pallas_call(
        body,
        in_specs=[pl.BlockSpec((NW, NP // D, D), lambda: (0, 0, 0))],
        out_specs=pl.BlockSpec((NP // D, D), lambda: (0, 0)),
        out_shape=jax.ShapeDtypeStruct((NP // D, D), jnp.float32),
    )(degp)


def _tc_stage0(dis, x, W1):
    def body(dis_ref, x_ref, w_ref, o_ref):
        xw = jnp.dot(x_ref[...], w_ref[...], preferred_element_type=jnp.float32)
        o_ref[...] = xw * dis_ref[...]

    return pl.pallas_call(
        body,
        grid=(N // _BLK,),
        in_specs=[
            pl.BlockSpec((_BLK, 1), lambda i: (i, 0)),
            pl.BlockSpec((_BLK, D), lambda i: (i, 0)),
            pl.BlockSpec((D, D), lambda i: (0, 0)),
        ],
        out_specs=pl.BlockSpec((_BLK, D), lambda i: (i, 0)),
        out_shape=jax.ShapeDtypeStruct((N, D), jnp.float32),
    )(dis, x, W1)


def _tc_stage1(dis, agg, y1, b1r, W2):
    def body(dis_ref, a_ref, y_ref, b_ref, w_ref, o_ref):
        dis_b = dis_ref[...]
        h = dis_b * (a_ref[0] + a_ref[1] + y_ref[...]) + b_ref[...]
        h = jnp.maximum(h, 0.0)
        o_ref[...] = dis_b * jnp.dot(h, w_ref[...],
                                     preferred_element_type=jnp.float32)

    return pl.pallas_call(
        body,
        grid=(N // _BLK,),
        in_specs=[
            pl.BlockSpec((_BLK, 1), lambda i: (i, 0)),
            pl.BlockSpec((NCORE, _BLK, D), lambda i: (0, i, 0)),
            pl.BlockSpec((_BLK, D), lambda i: (i, 0)),
            pl.BlockSpec((1, D), lambda i: (0, 0)),
            pl.BlockSpec((D, D), lambda i: (0, 0)),
        ],
        out_specs=pl.BlockSpec((_BLK, D), lambda i: (i, 0)),
        out_shape=jax.ShapeDtypeStruct((N, D), jnp.float32),
    )(dis, agg, y1, b1r, W2)


def _tc_stage2(dis, agg, y2, b2r, fcWp, fcbr):
    def body(dis_ref, a_ref, y_ref, b_ref, w_ref, fb_ref, o_ref):
        h = dis_ref[...] * (a_ref[0] + a_ref[1] + y_ref[...]) + b_ref[...]
        o = jnp.dot(h, w_ref[...], preferred_element_type=jnp.float32)
        o = o + fb_ref[...]
        o_ref[...] = jnp.where(o >= 0.0, o, 0.01 * o)

    return pl.pallas_call(
        body,
        grid=(N // _BLK,),
        in_specs=[
            pl.BlockSpec((_BLK, 1), lambda i: (i, 0)),
            pl.BlockSpec((NCORE, _BLK, D), lambda i: (0, i, 0)),
            pl.BlockSpec((_BLK, D), lambda i: (i, 0)),
            pl.BlockSpec((1, D), lambda i: (0, 0)),
            pl.BlockSpec((D, D), lambda i: (0, 0)),
            pl.BlockSpec((1, D), lambda i: (0, 0)),
        ],
        out_specs=pl.BlockSpec((_BLK, D), lambda i: (i, 0)),
        out_shape=jax.ShapeDtypeStruct((N, D), jnp.float32),
    )(dis, agg, y2, b2r, fcWp, fcbr)


def kernel(x, edge_index, W1, b1, W2, b2, fc_W, fc_b):
    src = edge_index[0].astype(jnp.int32)
    dst = edge_index[1].astype(jnp.int32)
    b1r = b1.reshape(1, D)
    b2r = b2.reshape(1, D)
    fcWp = jnp.zeros((D, D), jnp.float32).at[:, :2].set(fc_W)
    fcbr = jnp.zeros((1, D), jnp.float32).at[0, :2].set(fc_b)


    degp = _sc_degree(dst).reshape(NW, NP // D, D)
    dis = _tc_dis(degp).reshape(NP, 1)[:N]

    y1 = _tc_stage0(dis, x, W1)
    agg1 = _sc_aggregate(y1, src, dst)[:, :N]
    y2 = _tc_stage1(dis, agg1, y1, b1r, W2)
    agg2 = _sc_aggregate(y2, src, dst)[:, :N]
    out = _tc_stage2(dis, agg2, y2, b2r, fcWp, fcbr)
    return out[:, :2]
